# baseline (speedup 1.0000x reference)
import jax
import jax.numpy as jnp
from jax.experimental import pallas as pl
from jax.experimental.pallas import tpu as pltpu

BM = 128


def _fused_body(p_ref, x_ref, b1_ref, b2_ref, w1_hbm, w2_hbm, out_ref,
                w1_vmem, w2_vmem, sem1, sem2):
    i = pl.program_id(0)
    cp1 = pltpu.make_async_copy(w1_hbm, w1_vmem, sem1)
    cp2 = pltpu.make_async_copy(w2_hbm, w2_vmem, sem2)

    @pl.when(i == 0)
    def _start():
        cp1.start()
        cp2.start()

    pooled = jnp.dot(p_ref[...], x_ref[...],
                     preferred_element_type=jnp.float32)

    @pl.when(i == 0)
    def _wait1():
        cp1.wait()

    h = jnp.dot(pooled, w1_vmem[...], preferred_element_type=jnp.float32)
    h = h + b1_ref[...]
    h = 0.5 * h * (1.0 + jax.lax.erf(h * 0.7071067811865476))

    @pl.when(i == 0)
    def _wait2():
        cp2.wait()

    out = jnp.dot(h, w2_vmem[...], preferred_element_type=jnp.float32)
    out_ref[...] = out + b2_ref[...]


def kernel(x, W1, b1, W2, b2):
    n, c, d = x.shape
    h_dim = W1.shape[1]
    q = 256
    win = c // q
    m = n * q
    xf = x.reshape(m * win, d)
    rows = jax.lax.broadcasted_iota(jnp.int32, (BM, win * BM), 0)
    cols = jax.lax.broadcasted_iota(jnp.int32, (BM, win * BM), 1)
    pool_mat = jnp.where(cols // win == rows, 1.0 / win, 0.0).astype(jnp.float32)
    grid = (m // BM,)
    out = pl.pallas_call(
        _fused_body,
        grid=grid,
        in_specs=[
            pl.BlockSpec((BM, win * BM), lambda i: (0, 0)),
            pl.BlockSpec((BM * win, d), lambda i: (i, 0)),
            pl.BlockSpec((1, h_dim), lambda i: (0, 0)),
            pl.BlockSpec((1, d), lambda i: (0, 0)),
            pl.BlockSpec(memory_space=pltpu.MemorySpace.HBM),
            pl.BlockSpec(memory_space=pltpu.MemorySpace.HBM),
        ],
        out_specs=pl.BlockSpec((BM, d), lambda i: (i, 0)),
        out_shape=jax.ShapeDtypeStruct((m, d), jnp.float32),
        scratch_shapes=[
            pltpu.VMEM((d, h_dim), jnp.float32),
            pltpu.VMEM((h_dim, d), jnp.float32),
            pltpu.SemaphoreType.DMA,
            pltpu.SemaphoreType.DMA,
        ],
    )(pool_mat, xf, b1.reshape(1, h_dim), b2.reshape(1, d), W1, W2)
    return out
